# Initial kernel scaffold; baseline (speedup 1.0000x reference)
import jax
import jax.numpy as jnp
from jax.experimental import pallas as pl


def kernel(x, embs):
    raise NotImplementedError("write your pallas kernel here")


# speedup vs baseline: 1.0531x; 1.0531x over previous
import functools

import jax
import jax.numpy as jnp
from jax import lax
from jax.experimental import pallas as pl
from jax.experimental.pallas import tpu as pltpu
from jax.experimental.pallas import tpu_sc as plsc

NUM_FIELDS = 26
BATCH = 16384
EMBED_DIM = 32

TOTAL = BATCH * NUM_FIELDS
NUM_WORKERS = 32
PER_WORKER = TOTAL // NUM_WORKERS
SLAB = 128
SLABS_PER_CHUNK = 8
CHUNK = SLAB * SLABS_PER_CHUNK
NUM_CHUNKS = PER_WORKER // CHUNK
ROWS_PER_W = PER_WORKER // SLAB


def _make_gather():
  mesh = plsc.VectorSubcoreMesh(core_axis_name="c", subcore_axis_name="s")

  @functools.partial(
      pl.kernel,
      mesh=mesh,
      out_type=jax.ShapeDtypeStruct((TOTAL, EMBED_DIM), jnp.float32),
      scratch_types=[
          pltpu.VMEM((SLABS_PER_CHUNK, SLAB), jnp.int32),
          pltpu.VMEM((CHUNK, EMBED_DIM), jnp.float32),
          pltpu.SemaphoreType.DMA,
      ],
      compiler_params=pltpu.CompilerParams(use_tc_tiling_on_sc=False),
  )
  def gather_kernel(table_hbm, idx_hbm, out_hbm, idx_v, rows_v, sem):
    wid = lax.axis_index("s") * 2 + lax.axis_index("c")
    row_base = wid * ROWS_PER_W
    out_base = wid * PER_WORKER

    def chunk_body(g, _):
      pltpu.sync_copy(
          idx_hbm.at[pl.ds(row_base + g * SLABS_PER_CHUNK, SLABS_PER_CHUNK)],
          idx_v)
      copies = []
      for j in range(SLABS_PER_CHUNK):
        copies.append(
            pltpu.async_copy(
                table_hbm.at[idx_v.at[j]],
                rows_v.at[pl.ds(j * SLAB, SLAB)],
                sem))
      for c in copies:
        c.wait()
      pltpu.sync_copy(rows_v, out_hbm.at[pl.ds(out_base + g * CHUNK, CHUNK)])
      return ()

    lax.fori_loop(0, NUM_CHUNKS, chunk_body, (), unroll=False)

  return gather_kernel


_gather = _make_gather()


@jax.jit
def kernel(x, embs):
  idx2d = x.reshape(TOTAL // SLAB, SLAB)
  out_flat = _gather(embs, idx2d)
  return out_flat.reshape(BATCH, NUM_FIELDS * EMBED_DIM)


# speedup vs baseline: 1.0700x; 1.0161x over previous
import functools

import jax
import jax.numpy as jnp
from jax import lax
from jax.experimental import pallas as pl
from jax.experimental.pallas import tpu as pltpu
from jax.experimental.pallas import tpu_sc as plsc

NUM_FIELDS = 26
BATCH = 16384
EMBED_DIM = 32

TOTAL = BATCH * NUM_FIELDS
NUM_WORKERS = 32
PER_WORKER = TOTAL // NUM_WORKERS
SLAB = 128
SLABS_PER_CHUNK = 8
CHUNK = SLAB * SLABS_PER_CHUNK
NUM_CHUNKS = PER_WORKER // CHUNK
SLAB_ROWS_PER_W = PER_WORKER // SLAB
NBUF = 2


def _make_gather():
  mesh = plsc.VectorSubcoreMesh(core_axis_name="c", subcore_axis_name="s")

  @functools.partial(
      pl.kernel,
      mesh=mesh,
      out_type=jax.ShapeDtypeStruct((TOTAL, EMBED_DIM), jnp.float32),
      scratch_types=[
          pltpu.VMEM((NBUF, SLABS_PER_CHUNK, SLAB), jnp.int32),
          pltpu.VMEM((NBUF, CHUNK, EMBED_DIM), jnp.float32),
          pltpu.SemaphoreType.DMA,
          pltpu.SemaphoreType.DMA,
      ],
      compiler_params=pltpu.CompilerParams(use_tc_tiling_on_sc=False),
  )
  def gather_kernel(table_hbm, idx_hbm, out_hbm, idx_v, rows_v, sem0, sem1):
    sems = (sem0, sem1)
    wid = lax.axis_index("s") * 2 + lax.axis_index("c")
    row_base = wid * SLAB_ROWS_PER_W
    out_base = wid * PER_WORKER

    def fire(g, b):
      pltpu.sync_copy(
          idx_hbm.at[pl.ds(row_base + g * SLABS_PER_CHUNK, SLABS_PER_CHUNK)],
          idx_v.at[b])
      for j in range(SLABS_PER_CHUNK):
        pltpu.async_copy(
            table_hbm.at[idx_v.at[b].at[j]],
            rows_v.at[b].at[pl.ds(j * SLAB, SLAB)],
            sems[b])

    def drain_and_writeback(g, b):
      pltpu.make_async_copy(
          out_hbm.at[pl.ds(0, CHUNK)], rows_v.at[b], sems[b]).wait()
      pltpu.sync_copy(
          rows_v.at[b], out_hbm.at[pl.ds(out_base + g * CHUNK, CHUNK)])

    fire(0, 0)
    fire(1, 1)

    def body(k, _):
      for b in range(NBUF):
        g = NBUF * k + b
        drain_and_writeback(g, b)

        @pl.when(g + NBUF < NUM_CHUNKS)
        def _():
          fire(g + NBUF, b)
      return ()

    lax.fori_loop(0, NUM_CHUNKS // NBUF, body, (), unroll=False)
    drain_and_writeback(NUM_CHUNKS - 1, (NUM_CHUNKS - 1) % NBUF)

  return gather_kernel


_gather = _make_gather()


@jax.jit
def kernel(x, embs):
  idx2d = x.reshape(TOTAL // SLAB, SLAB)
  out_flat = _gather(embs, idx2d)
  return out_flat.reshape(BATCH, NUM_FIELDS * EMBED_DIM)
